# baseline (speedup 1.0000x reference)
import functools

import jax
import jax.numpy as jnp
from jax import lax
from jax.experimental import pallas as pl
from jax.experimental.pallas import tpu as pltpu
from jax.experimental.pallas import tpu_sc as plsc

_L = 16
_NC = 2
_NS = 16
_NW = _NC * _NS
_OUT_CHUNK = 4096


def _sc_gather_body(mus_hbm, sig_hbm, z_hbm, muz_hbm, sigz_hbm,
                    z_v, row_v, out_v, sem0, sem1):
    D, N = mus_hbm.shape
    B = z_hbm.shape[0]
    rows_per_w = D // _NW

    wid = lax.axis_index("s") * _NC + lax.axis_index("c")

    pltpu.sync_copy(z_hbm, z_v)

    n_chunks = B // _OUT_CHUNK
    sems = (sem0, sem1)
    pending = [None, None]

    for src, dst in ((mus_hbm, muz_hbm), (sig_hbm, sigz_hbm)):
        for r in range(rows_per_w):
            d = wid * rows_per_w + r
            pltpu.sync_copy(src.at[d], row_v)
            for h in range(n_chunks):
                b = h % 2
                if pending[b] is not None:
                    pending[b].wait()
                    pending[b] = None

                @plsc.parallel_loop(0, _OUT_CHUNK, step=_L, unroll=8)
                def gather_step(j, h=h, b=b):
                    idx = z_v[pl.ds(h * _OUT_CHUNK + j, _L)]
                    out_v[b, pl.ds(j, _L)] = plsc.load_gather(row_v, [idx])

                pending[b] = pltpu.async_copy(
                    out_v.at[b],
                    dst.at[d, pl.ds(h * _OUT_CHUNK, _OUT_CHUNK)],
                    sems[b],
                )
    for b in range(2):
        if pending[b] is not None:
            pending[b].wait()


def kernel(mus, sigmas, z):
    D, N = mus.shape
    B = z.shape[0]
    out = jax.ShapeDtypeStruct((D, B), jnp.float32)
    mesh = plsc.VectorSubcoreMesh(core_axis_name="c", subcore_axis_name="s")
    k = functools.partial(
        pl.kernel,
        out_type=(out, out),
        mesh=mesh,
        scratch_types=[
            pltpu.VMEM((B,), jnp.int32),
            pltpu.VMEM((N,), jnp.float32),
            pltpu.VMEM((2, _OUT_CHUNK), jnp.float32),
            pltpu.SemaphoreType.DMA,
            pltpu.SemaphoreType.DMA,
        ],
        compiler_params=pltpu.CompilerParams(
            needs_layout_passes=False, use_tc_tiling_on_sc=True),
    )(_sc_gather_body)
    return k(mus, sigmas, z)


# speedup vs baseline: 3.0064x; 3.0064x over previous
import functools

import jax
import jax.numpy as jnp
from jax import lax
from jax.experimental import pallas as pl
from jax.experimental.pallas import tpu as pltpu
from jax.experimental.pallas import tpu_sc as plsc

_NC = 2
_NS = 16
_NW = _NC * _NS
_CHUNK = 128


def _sc_rowgather_body(mus_hbm, sig_hbm, z_hbm, muz_hbm, sigz_hbm,
                       z_v, rows_v, gs0, gs1, ss0, ss1):
    N, D = mus_hbm.shape
    B = z_hbm.shape[0]
    b_per_w = B // _NW
    n_g = b_per_w // _CHUNK

    wid = lax.axis_index("s") * _NC + lax.axis_index("c")
    base = wid * b_per_w

    for g in range(n_g):
        pltpu.sync_copy(z_hbm.at[pl.ds(base + g * _CHUNK, _CHUNK)], z_v.at[g])

    gsems = (gs0, gs1)
    ssems = (ss0, ss1)
    items = [(src, dst, g)
             for src, dst in ((mus_hbm, muz_hbm), (sig_hbm, sigz_hbm))
             for g in range(n_g)]
    n = len(items)
    pend_g = [None, None]
    pend_s = [None, None]

    def issue_gather(i, b):
        src, _, g = items[i]
        pend_g[b] = pltpu.async_copy(src.at[z_v.at[g]], rows_v.at[b], gsems[b])

    issue_gather(0, 0)
    for i in range(n):
        b = i % 2
        if i + 1 < n:
            b2 = (i + 1) % 2
            if pend_s[b2] is not None:
                pend_s[b2].wait()
                pend_s[b2] = None
            issue_gather(i + 1, b2)
        pend_g[b].wait()
        _, dst, g = items[i]
        pend_s[b] = pltpu.async_copy(
            rows_v.at[b], dst.at[pl.ds(base + g * _CHUNK, _CHUNK)], ssems[b])
    for b in range(2):
        if pend_s[b] is not None:
            pend_s[b].wait()


def kernel(mus, sigmas, z):
    D, N = mus.shape
    B = z.shape[0]
    mus_t = mus.T
    sig_t = sigmas.T
    out_t = jax.ShapeDtypeStruct((B, D), jnp.float32)
    mesh = plsc.VectorSubcoreMesh(core_axis_name="c", subcore_axis_name="s")
    b_per_w = B // _NW
    n_g = b_per_w // _CHUNK
    k = functools.partial(
        pl.kernel,
        out_type=(out_t, out_t),
        mesh=mesh,
        scratch_types=[
            pltpu.VMEM((n_g, _CHUNK), jnp.int32),
            pltpu.VMEM((2, _CHUNK, D), jnp.float32),
            pltpu.SemaphoreType.DMA,
            pltpu.SemaphoreType.DMA,
            pltpu.SemaphoreType.DMA,
            pltpu.SemaphoreType.DMA,
        ],
        compiler_params=pltpu.CompilerParams(needs_layout_passes=False),
    )(_sc_rowgather_body)
    muz_t, sigz_t = k(mus_t, sig_t, z)
    return muz_t.T, sigz_t.T
